# Initial kernel scaffold; baseline (speedup 1.0000x reference)
import jax
import jax.numpy as jnp
from jax.experimental import pallas as pl


def kernel(seq, segment_label, token_table, position_table, segment_table, ln_gamma, ln_beta):
    raise NotImplementedError("write your pallas kernel here")


# speedup vs baseline: 10.0501x; 10.0501x over previous
import functools

import jax
import jax.numpy as jnp
from jax import lax
from jax.experimental import pallas as pl
from jax.experimental.pallas import tpu as pltpu
from jax.experimental.pallas import tpu_sc as plsc

VOCAB = 100000
D = 128
BATCH = 1024
SEQ = 200
ROWS = BATCH * SEQ

_INFO = plsc.get_sparse_core_info()
_NC = _INFO.num_cores
_NS = _INFO.num_subcores
_NW = _NC * _NS

CH = 128
NCHT = ROWS // CH
NCH = NCHT // _NW
RPW = ROWS // _NW

_SC_MESH = plsc.VectorSubcoreMesh(core_axis_name="c", subcore_axis_name="s")


NSTG = NCH + 6


@functools.partial(
    pl.kernel,
    mesh=_SC_MESH,
    out_type=jax.ShapeDtypeStruct((ROWS, D), jnp.float32),
    scratch_types=[
        pltpu.VMEM((NSTG, CH), jnp.int32),
        pltpu.VMEM((CH, D), jnp.float32),
        pltpu.SemaphoreType.DMA,
    ],
)
def _sc_gather(idx_hbm, table_hbm, out_hbm, idx_v, buf, sem):
    wid = lax.axis_index("s") * _NC + lax.axis_index("c")
    base_chunk = wid * NCH
    start8 = pl.multiple_of((base_chunk // 8) * 8, 8)
    skew = base_chunk - start8
    pltpu.sync_copy(idx_hbm.at[pl.ds(start8, NSTG)], idx_v)

    def body(ci, _):
        row_off = pl.multiple_of((base_chunk + ci) * CH, CH)
        pltpu.async_copy(table_hbm.at[idx_v.at[skew + ci]], buf, sem).wait()
        pltpu.sync_copy(buf, out_hbm.at[pl.ds(row_off, CH)])
        return 0

    lax.fori_loop(0, NCH, body, 0, unroll=False)


BB = 32


def _tc_body(tok_ref, lbl_ref, pos_ref, seg_ref, gamma_ref, beta_ref, out_ref):
    tok = tok_ref[...]
    lbl = lbl_ref[...]
    pos = pos_ref[...]
    seg_tab = seg_ref[...]
    x = tok + pos[None, :, :]
    l3 = lbl[:, :, None]
    seg = jnp.where(
        l3 == 1,
        seg_tab[1][None, None, :],
        jnp.where(l3 == 2, seg_tab[2][None, None, :], seg_tab[0][None, None, :]),
    )
    x = x + seg
    mean = jnp.mean(x, axis=-1, keepdims=True)
    xc = x - mean
    var = jnp.mean(xc * xc, axis=-1, keepdims=True)
    xhat = xc * lax.rsqrt(var + 1e-12)
    out_ref[...] = xhat * gamma_ref[...][None, None, :] + beta_ref[...][None, None, :]


def kernel(seq, segment_label, token_table, position_table, segment_table, ln_gamma, ln_beta):
    idx2d = seq.reshape(NCHT, CH).astype(jnp.int32)
    gathered = _sc_gather(idx2d, token_table)
    tok3 = gathered.reshape(BATCH, SEQ, D)
    pos = position_table[:SEQ]

    out = pl.pallas_call(
        _tc_body,
        grid=(BATCH // BB,),
        in_specs=[
            pl.BlockSpec((BB, SEQ, D), lambda i: (i, 0, 0)),
            pl.BlockSpec((BB, SEQ), lambda i: (i, 0)),
            pl.BlockSpec((SEQ, D), lambda i: (0, 0)),
            pl.BlockSpec((3, D), lambda i: (0, 0)),
            pl.BlockSpec((D,), lambda i: (0,)),
            pl.BlockSpec((D,), lambda i: (0,)),
        ],
        out_specs=pl.BlockSpec((BB, SEQ, D), lambda i: (i, 0, 0)),
        out_shape=jax.ShapeDtypeStruct((BATCH, SEQ, D), jnp.float32),
    )(tok3, segment_label, pos, segment_table, ln_gamma, ln_beta)
    return out


# speedup vs baseline: 10.8766x; 1.0822x over previous
import functools

import jax
import jax.numpy as jnp
from jax import lax
from jax.experimental import pallas as pl
from jax.experimental.pallas import tpu as pltpu
from jax.experimental.pallas import tpu_sc as plsc

VOCAB = 100000
D = 128
BATCH = 1024
SEQ = 200
ROWS = BATCH * SEQ

_INFO = plsc.get_sparse_core_info()
_NC = _INFO.num_cores
_NS = _INFO.num_subcores
_NW = _NC * _NS

CH = 128
NCHT = ROWS // CH
NCH = NCHT // _NW
RPW = ROWS // _NW

_SC_MESH = plsc.VectorSubcoreMesh(core_axis_name="c", subcore_axis_name="s")


NSTG = NCH + 6


NP = NCH // 2


@functools.partial(
    pl.kernel,
    mesh=_SC_MESH,
    out_type=jax.ShapeDtypeStruct((ROWS, D), jnp.float32),
    scratch_types=[
        pltpu.VMEM((NSTG, CH), jnp.int32),
        pltpu.VMEM((CH, D), jnp.float32),
        pltpu.VMEM((CH, D), jnp.float32),
        pltpu.SemaphoreType.DMA,
        pltpu.SemaphoreType.DMA,
        pltpu.SemaphoreType.DMA,
        pltpu.SemaphoreType.DMA,
    ],
)
def _sc_gather(idx_hbm, table_hbm, out_hbm, idx_v, buf0, buf1, sg0, sg1, ss0, ss1):
    wid = lax.axis_index("s") * _NC + lax.axis_index("c")
    base_chunk = wid * NCH
    start8 = pl.multiple_of((base_chunk // 8) * 8, 8)
    skew = base_chunk - start8
    pltpu.sync_copy(idx_hbm.at[pl.ds(start8, NSTG)], idx_v)

    def _gather(ci, buf, sem):
        return pltpu.make_async_copy(table_hbm.at[idx_v.at[skew + ci]], buf, sem)

    def _store(ci, buf, sem):
        row_off = pl.multiple_of((base_chunk + ci) * CH, CH)
        return pltpu.make_async_copy(buf, out_hbm.at[pl.ds(row_off, CH)], sem)

    _gather(0, buf0, sg0).start()

    def pair(p, _):
        c0 = 2 * p
        _gather(c0, buf0, sg0).wait()

        @pl.when(p > 0)
        def _():
            _store(c0 - 1, buf1, ss1).wait()

        _gather(c0 + 1, buf1, sg1).start()
        _store(c0, buf0, ss0).start()

        _gather(c0 + 1, buf1, sg1).wait()

        @pl.when(p < NP - 1)
        def _():
            _store(c0, buf0, ss0).wait()
            _gather(c0 + 2, buf0, sg0).start()

        _store(c0 + 1, buf1, ss1).start()
        return 0

    lax.fori_loop(0, NP, pair, 0, unroll=False)
    _store(NCH - 2, buf0, ss0).wait()
    _store(NCH - 1, buf1, ss1).wait()


BB = 32


def _tc_body(tok_ref, lbl_ref, pos_ref, seg_ref, gamma_ref, beta_ref, out_ref):
    tok = tok_ref[...]
    lbl = lbl_ref[...]
    pos = pos_ref[...]
    seg_tab = seg_ref[...]
    x = tok + pos[None, :, :]
    l3 = lbl[:, :, None]
    seg = jnp.where(
        l3 == 1,
        seg_tab[1][None, None, :],
        jnp.where(l3 == 2, seg_tab[2][None, None, :], seg_tab[0][None, None, :]),
    )
    x = x + seg
    mean = jnp.mean(x, axis=-1, keepdims=True)
    xc = x - mean
    var = jnp.mean(xc * xc, axis=-1, keepdims=True)
    xhat = xc * lax.rsqrt(var + 1e-12)
    out_ref[...] = xhat * gamma_ref[...][None, None, :] + beta_ref[...][None, None, :]


def kernel(seq, segment_label, token_table, position_table, segment_table, ln_gamma, ln_beta):
    idx2d = seq.reshape(NCHT, CH).astype(jnp.int32)
    gathered = _sc_gather(idx2d, token_table)
    tok3 = gathered.reshape(BATCH, SEQ, D)
    pos = position_table[:SEQ]

    out = pl.pallas_call(
        _tc_body,
        grid=(BATCH // BB,),
        in_specs=[
            pl.BlockSpec((BB, SEQ, D), lambda i: (i, 0, 0)),
            pl.BlockSpec((BB, SEQ), lambda i: (i, 0)),
            pl.BlockSpec((SEQ, D), lambda i: (0, 0)),
            pl.BlockSpec((3, D), lambda i: (0, 0)),
            pl.BlockSpec((D,), lambda i: (0,)),
            pl.BlockSpec((D,), lambda i: (0,)),
        ],
        out_specs=pl.BlockSpec((BB, SEQ, D), lambda i: (i, 0, 0)),
        out_shape=jax.ShapeDtypeStruct((BATCH, SEQ, D), jnp.float32),
    )(tok3, segment_label, pos, segment_table, ln_gamma, ln_beta)
    return out
